# Initial kernel scaffold; baseline (speedup 1.0000x reference)
import jax
import jax.numpy as jnp
from jax.experimental import pallas as pl


def kernel(node_features, edge_features, edge_index, W_e, b_e, g_e, bt_e, W_n, b_n, g_n, bt_n):
    raise NotImplementedError("write your pallas kernel here")


# speedup vs baseline: 6.6222x; 6.6222x over previous
import functools

import jax
import jax.numpy as jnp
from jax import lax
from jax.experimental import pallas as pl
from jax.experimental.pallas import tpu as pltpu
from jax.experimental.pallas import tpu_sc as plsc

NC = 2
NS = 16
NW = NC * NS


def _proj_body(nf_ref, ws_ref, wt_ref, ps_ref, pt_ref):
    x = nf_ref[...]
    ps_ref[...] = jnp.dot(x, ws_ref[...], preferred_element_type=jnp.float32)
    pt_ref[...] = jnp.dot(x, wt_ref[...], preferred_element_type=jnp.float32)


def _tc_project(nf, w_s, w_t):
    n, _ = nf.shape
    de = w_s.shape[1]
    out = jax.ShapeDtypeStruct((n, de), jnp.float32)
    return pl.pallas_call(_proj_body, out_shape=(out, out))(nf, w_s, w_t)


def _sc_gather(ps, pt, src, tgt, epw, ch):
    n, de = ps.shape
    e = src.shape[0]
    nchunk = epw // ch
    mesh = plsc.VectorSubcoreMesh(core_axis_name="c", subcore_axis_name="s")

    @functools.partial(
        pl.kernel,
        out_type=jax.ShapeDtypeStruct((e, de), jnp.float32),
        mesh=mesh,
        compiler_params=pltpu.CompilerParams(use_tc_tiling_on_sc=False),
        scratch_types=[
            pltpu.VMEM_SHARED((n, de), jnp.float32),
            pltpu.VMEM_SHARED((n, de), jnp.float32),
            pltpu.VMEM((ch,), jnp.int32),
            pltpu.VMEM((ch,), jnp.int32),
            pltpu.VMEM((ch, de), jnp.float32),
            pltpu.VMEM((ch, de), jnp.float32),
        ],
    )
    def k(ps_hbm, pt_hbm, src_hbm, tgt_hbm, out_hbm, ps_sh, pt_sh, idx_s, idx_t, gs, gt):
        cid = lax.axis_index("c")
        sid = lax.axis_index("s")
        wid = cid * NS + sid

        @pl.when(sid == 0)
        def _():
            pltpu.sync_copy(ps_hbm, ps_sh)
            pltpu.sync_copy(pt_hbm, pt_sh)

        plsc.subcore_barrier()

        def chunk(ci, carry):
            base = wid * epw + ci * ch
            pltpu.sync_copy(src_hbm.at[pl.ds(base, ch)], idx_s)
            pltpu.sync_copy(tgt_hbm.at[pl.ds(base, ch)], idx_t)
            pltpu.sync_copy(ps_sh.at[idx_s], gs)
            pltpu.sync_copy(pt_sh.at[idx_t], gt)

            def row(i, c2):
                gs[i, :] = gs[i, :] + gt[i, :]
                return c2

            lax.fori_loop(0, ch, row, 0, unroll=8)
            pltpu.sync_copy(gs, out_hbm.at[pl.ds(base, ch)])
            return carry

        lax.fori_loop(0, nchunk, chunk, 0)

    return k(ps, pt, src, tgt)


def _edge_body(ef_ref, g_ref, wee_ref, be_ref, ge_ref, bte_ref, out_ref):
    ef = ef_ref[...]
    x = ef + jnp.dot(ef, wee_ref[...], preferred_element_type=jnp.float32)
    x = x + g_ref[...] + be_ref[...]
    mu = jnp.mean(x, axis=-1, keepdims=True)
    xc = x - mu
    var = jnp.mean(xc * xc, axis=-1, keepdims=True)
    out_ref[...] = xc * lax.rsqrt(var + 1e-5) * ge_ref[...] + bte_ref[...]


def _tc_edge_update(ef, g, w_ee, b_e, g_e, bt_e, be):
    e, de = ef.shape
    grid = e // be
    row_spec = pl.BlockSpec((be, de), lambda i: (i, 0))
    par_spec = pl.BlockSpec((1, de), lambda i: (0, 0))
    w_spec = pl.BlockSpec((de, de), lambda i: (0, 0))
    return pl.pallas_call(
        _edge_body,
        grid=(grid,),
        in_specs=[row_spec, row_spec, w_spec, par_spec, par_spec, par_spec],
        out_specs=row_spec,
        out_shape=jax.ShapeDtypeStruct((e, de), jnp.float32),
    )(ef, g, w_ee, b_e.reshape(1, de), g_e.reshape(1, de), bt_e.reshape(1, de))


def _sc_scatter(new_edges, src, tgt, n, epw, ch):
    e, de = new_edges.shape
    nchunk = epw // ch
    mesh = plsc.VectorSubcoreMesh(core_axis_name="c", subcore_axis_name="s")
    zr = n // NS
    part = jax.ShapeDtypeStruct((NC, n, de), jnp.float32)

    @functools.partial(
        pl.kernel,
        out_type=(part, part),
        mesh=mesh,
        compiler_params=pltpu.CompilerParams(use_tc_tiling_on_sc=False),
        scratch_types=[
            pltpu.VMEM_SHARED((n, de), jnp.float32),
            pltpu.VMEM_SHARED((n, de), jnp.float32),
            pltpu.VMEM((ch,), jnp.int32),
            pltpu.VMEM((ch,), jnp.int32),
            pltpu.VMEM((ch, de), jnp.float32),
            pltpu.VMEM((ch, de), jnp.float32),
        ],
    )
    def k(ne_hbm, src_hbm, tgt_hbm, zeros_hbm, ones_hbm,
          sum_hbm, cnt_hbm, acc_s, acc_c, idx_s, idx_t, val, ones_v):
        cid = lax.axis_index("c")
        sid = lax.axis_index("s")
        wid = cid * NS + sid

        pltpu.sync_copy(ones_hbm, ones_v)
        pltpu.sync_copy(zeros_hbm.at[pl.ds(sid * zr, zr)], acc_s.at[pl.ds(sid * zr, zr)])
        pltpu.sync_copy(zeros_hbm.at[pl.ds(sid * zr, zr)], acc_c.at[pl.ds(sid * zr, zr)])
        plsc.subcore_barrier()

        def chunk(ci, carry):
            base = wid * epw + ci * ch
            pltpu.sync_copy(src_hbm.at[pl.ds(base, ch)], idx_s)
            pltpu.sync_copy(tgt_hbm.at[pl.ds(base, ch)], idx_t)
            pltpu.sync_copy(ne_hbm.at[pl.ds(base, ch)], val)
            pltpu.sync_copy(val, acc_s.at[idx_s], add=True)
            pltpu.sync_copy(val, acc_s.at[idx_t], add=True)
            pltpu.sync_copy(ones_v, acc_c.at[idx_s], add=True)
            pltpu.sync_copy(ones_v, acc_c.at[idx_t], add=True)
            return carry

        lax.fori_loop(0, nchunk, chunk, 0)
        plsc.subcore_barrier()
        pltpu.sync_copy(acc_s.at[pl.ds(sid * zr, zr)], sum_hbm.at[cid, pl.ds(sid * zr, zr)])
        pltpu.sync_copy(acc_c.at[pl.ds(sid * zr, zr)], cnt_hbm.at[cid, pl.ds(sid * zr, zr)])

    zeros = jnp.zeros((n, de), jnp.float32)
    ones = jnp.ones((ch, de), jnp.float32)
    return k(new_edges, src, tgt, zeros, ones)


def _node_body(nf_ref, sum_ref, cnt_ref, wn1_ref, wn2_ref, bn_ref, gn_ref,
               btn_ref, out_ref):
    nf = nf_ref[...]
    s = sum_ref[0] + sum_ref[1]
    c = cnt_ref[0, :, 0:1] + cnt_ref[1, :, 0:1]
    m = s / (c + 1e-10)
    x = nf + jnp.dot(nf, wn1_ref[...], preferred_element_type=jnp.float32)
    x = x + jnp.dot(m, wn2_ref[...], preferred_element_type=jnp.float32)
    x = x + bn_ref[...]
    mu = jnp.mean(x, axis=-1, keepdims=True)
    xc = x - mu
    var = jnp.mean(xc * xc, axis=-1, keepdims=True)
    out_ref[...] = xc * lax.rsqrt(var + 1e-5) * gn_ref[...] + btn_ref[...]


def _tc_node_update(nf, sum_p, cnt_p, w_n1, w_n2, b_n, g_n, bt_n):
    n, dn = nf.shape
    return pl.pallas_call(
        _node_body,
        out_shape=jax.ShapeDtypeStruct((n, dn), jnp.float32),
    )(nf, sum_p, cnt_p, w_n1, w_n2,
      b_n.reshape(1, dn), g_n.reshape(1, dn), bt_n.reshape(1, dn))


def kernel(node_features, edge_features, edge_index, W_e, b_e, g_e, bt_e,
           W_n, b_n, g_n, bt_n):
    n, dn = node_features.shape
    e, de = edge_features.shape
    src = edge_index[0]
    tgt = edge_index[1]
    w_ee = W_e[:de]
    w_es = W_e[de:de + dn]
    w_et = W_e[de + dn:]
    w_n1 = W_n[:dn]
    w_n2 = W_n[dn:]

    epw = e // NW
    ch = 2000

    ps, pt = _tc_project(node_features, w_es, w_et)
    g = _sc_gather(ps, pt, src, tgt, epw, ch)
    new_edges = _tc_edge_update(edge_features, g, w_ee, b_e, g_e, bt_e, be=8000)
    sum_p, cnt_p = _sc_scatter(new_edges, src, tgt, n, epw, ch)
    new_nodes = _tc_node_update(node_features, sum_p, cnt_p, w_n1, w_n2,
                                b_n, g_n, bt_n)
    return (new_nodes, new_edges)


# speedup vs baseline: 11.3555x; 1.7148x over previous
import functools

import jax
import jax.numpy as jnp
from jax import lax
from jax.experimental import pallas as pl
from jax.experimental.pallas import tpu as pltpu
from jax.experimental.pallas import tpu_sc as plsc

NC = 2
NS = 16
NW = NC * NS
PK = 8


def _proj_body(nf_ref, ws_ref, wt_ref, ps_ref, pt_ref):
    x = nf_ref[...]
    ps_ref[...] = jnp.dot(x, ws_ref[...], preferred_element_type=jnp.float32)
    pt_ref[...] = jnp.dot(x, wt_ref[...], preferred_element_type=jnp.float32)


def _tc_project(nf, w_s, w_t):
    n, _ = nf.shape
    de = w_s.shape[1]
    out = jax.ShapeDtypeStruct((n, de), jnp.float32)
    return pl.pallas_call(_proj_body, out_shape=(out, out))(nf, w_s, w_t)


def _sc_gather(ps, pt, src, tgt, epw, ch):
    n, de = ps.shape
    e = src.shape[0]
    ep, dep = e // PK, de * PK
    nchunk = epw // ch
    chp = ch // PK
    mesh = plsc.VectorSubcoreMesh(core_axis_name="c", subcore_axis_name="s")

    @functools.partial(
        pl.kernel,
        out_type=jax.ShapeDtypeStruct((ep, dep), jnp.float32),
        mesh=mesh,
        compiler_params=pltpu.CompilerParams(use_tc_tiling_on_sc=False),
        scratch_types=[
            pltpu.VMEM_SHARED((n, de), jnp.float32),
            pltpu.VMEM_SHARED((n, de), jnp.float32),
            pltpu.VMEM((ch,), jnp.int32),
            pltpu.VMEM((ch,), jnp.int32),
            pltpu.VMEM((ch, de), jnp.float32),
            pltpu.VMEM((ch, de), jnp.float32),
            pltpu.VMEM((chp, dep), jnp.float32),
        ],
    )
    def k(ps_hbm, pt_hbm, src_hbm, tgt_hbm, out_hbm,
          ps_sh, pt_sh, idx_s, idx_t, gs, gt, xp):
        cid = lax.axis_index("c")
        sid = lax.axis_index("s")
        wid = cid * NS + sid

        @pl.when(sid == 0)
        def _():
            pltpu.sync_copy(ps_hbm, ps_sh)
            pltpu.sync_copy(pt_hbm, pt_sh)

        plsc.subcore_barrier()

        def chunk(ci, carry):
            base = wid * epw + ci * ch
            pbase = base // PK
            pltpu.sync_copy(src_hbm.at[pl.ds(base, ch)], idx_s)
            pltpu.sync_copy(tgt_hbm.at[pl.ds(base, ch)], idx_t)
            pltpu.sync_copy(ps_sh.at[idx_s], gs)
            pltpu.sync_copy(pt_sh.at[idx_t], gt)

            def row(i, c2):
                xp[i // PK, pl.ds((i % PK) * de, de)] = gs[i, :] + gt[i, :]
                return c2

            lax.fori_loop(0, ch, row, 0, unroll=8)
            pltpu.sync_copy(xp, out_hbm.at[pl.ds(pbase, chp)])
            return carry

        lax.fori_loop(0, nchunk, chunk, 0)

    return k(ps, pt, src, tgt)


def _ln_body(efp_ref, gp_ref, wblk_ref, bd_ref, bblk_ref, ge_ref, bte_ref, out_ref):
    x = jnp.dot(efp_ref[...], wblk_ref[...], preferred_element_type=jnp.float32)
    x = x + gp_ref[...] + bblk_ref[...]
    mu = jnp.dot(x, bd_ref[...], preferred_element_type=jnp.float32)
    xc = x - mu
    var = jnp.dot(xc * xc, bd_ref[...], preferred_element_type=jnp.float32)
    out_ref[...] = xc * lax.rsqrt(var + 1e-5) * ge_ref[...] + bte_ref[...]


def _tc_edge_ln(efp, gp, wblk, bd, bblk, g_e, bt_e, bp):
    ep, dep = efp.shape
    grid = ep // bp
    row_spec = pl.BlockSpec((bp, dep), lambda i: (i, 0))
    par_spec = pl.BlockSpec((1, dep), lambda i: (0, 0))
    mat_spec = pl.BlockSpec((dep, dep), lambda i: (0, 0))
    return pl.pallas_call(
        _ln_body,
        grid=(grid,),
        in_specs=[row_spec, row_spec, mat_spec, mat_spec,
                  par_spec, par_spec, par_spec],
        out_specs=row_spec,
        out_shape=jax.ShapeDtypeStruct((ep, dep), jnp.float32),
    )(efp, gp, wblk, bd, bblk.reshape(1, dep),
      jnp.tile(g_e, (PK,)).reshape(1, dep),
      jnp.tile(bt_e, (PK,)).reshape(1, dep))


def _sc_scatter(nep, src, tgt, n, de, epw, ch):
    ep, dep = nep.shape
    nchunk = epw // ch
    chp = ch // PK
    mesh = plsc.VectorSubcoreMesh(core_axis_name="c", subcore_axis_name="s")
    zr = n // NS
    part = jax.ShapeDtypeStruct((NC, n, de), jnp.float32)

    @functools.partial(
        pl.kernel,
        out_type=(part, part),
        mesh=mesh,
        compiler_params=pltpu.CompilerParams(use_tc_tiling_on_sc=False),
        scratch_types=[
            pltpu.VMEM_SHARED((n, de), jnp.float32),
            pltpu.VMEM_SHARED((n, de), jnp.float32),
            pltpu.VMEM((ch,), jnp.int32),
            pltpu.VMEM((ch,), jnp.int32),
            pltpu.VMEM((chp, dep), jnp.float32),
            pltpu.VMEM((ch, de), jnp.float32),
            pltpu.VMEM((ch, de), jnp.float32),
        ],
    )
    def k(ne_hbm, src_hbm, tgt_hbm, zeros_hbm, ones_hbm,
          sum_hbm, cnt_hbm, acc_s, acc_c, idx_s, idx_t, vp, val, ones_v):
        cid = lax.axis_index("c")
        sid = lax.axis_index("s")
        wid = cid * NS + sid

        pltpu.sync_copy(ones_hbm, ones_v)
        pltpu.sync_copy(zeros_hbm.at[pl.ds(sid * zr, zr)], acc_s.at[pl.ds(sid * zr, zr)])
        pltpu.sync_copy(zeros_hbm.at[pl.ds(sid * zr, zr)], acc_c.at[pl.ds(sid * zr, zr)])
        plsc.subcore_barrier()

        def chunk(ci, carry):
            base = wid * epw + ci * ch
            pbase = base // PK
            pltpu.sync_copy(src_hbm.at[pl.ds(base, ch)], idx_s)
            pltpu.sync_copy(tgt_hbm.at[pl.ds(base, ch)], idx_t)
            pltpu.sync_copy(ne_hbm.at[pl.ds(pbase, chp)], vp)

            def row(i, c2):
                val[i, :] = vp[i // PK, pl.ds((i % PK) * de, de)]
                return c2

            lax.fori_loop(0, ch, row, 0, unroll=8)
            pltpu.sync_copy(val, acc_s.at[idx_s], add=True)
            pltpu.sync_copy(val, acc_s.at[idx_t], add=True)
            pltpu.sync_copy(ones_v, acc_c.at[idx_s], add=True)
            pltpu.sync_copy(ones_v, acc_c.at[idx_t], add=True)
            return carry

        lax.fori_loop(0, nchunk, chunk, 0)
        plsc.subcore_barrier()
        pltpu.sync_copy(acc_s.at[pl.ds(sid * zr, zr)], sum_hbm.at[cid, pl.ds(sid * zr, zr)])
        pltpu.sync_copy(acc_c.at[pl.ds(sid * zr, zr)], cnt_hbm.at[cid, pl.ds(sid * zr, zr)])

    zeros = jnp.zeros((n, de), jnp.float32)
    ones = jnp.ones((ch, de), jnp.float32)
    return k(nep, src, tgt, zeros, ones)


def _node_body(nf_ref, sum_ref, cnt_ref, wn1_ref, wn2_ref, bn_ref, gn_ref,
               btn_ref, out_ref):
    nf = nf_ref[...]
    s = sum_ref[0] + sum_ref[1]
    c = cnt_ref[0, :, 0:1] + cnt_ref[1, :, 0:1]
    m = s / (c + 1e-10)
    x = nf + jnp.dot(nf, wn1_ref[...], preferred_element_type=jnp.float32)
    x = x + jnp.dot(m, wn2_ref[...], preferred_element_type=jnp.float32)
    x = x + bn_ref[...]
    mu = jnp.mean(x, axis=-1, keepdims=True)
    xc = x - mu
    var = jnp.mean(xc * xc, axis=-1, keepdims=True)
    out_ref[...] = xc * lax.rsqrt(var + 1e-5) * gn_ref[...] + btn_ref[...]


def _tc_node_update(nf, sum_p, cnt_p, w_n1, w_n2, b_n, g_n, bt_n):
    n, dn = nf.shape
    return pl.pallas_call(
        _node_body,
        out_shape=jax.ShapeDtypeStruct((n, dn), jnp.float32),
    )(nf, sum_p, cnt_p, w_n1, w_n2,
      b_n.reshape(1, dn), g_n.reshape(1, dn), bt_n.reshape(1, dn))


def kernel(node_features, edge_features, edge_index, W_e, b_e, g_e, bt_e,
           W_n, b_n, g_n, bt_n):
    n, dn = node_features.shape
    e, de = edge_features.shape
    src = edge_index[0]
    tgt = edge_index[1]
    w_ee = W_e[:de]
    w_es = W_e[de:de + dn]
    w_et = W_e[de + dn:]
    w_n1 = W_n[:dn]
    w_n2 = W_n[dn:]

    epw = e // NW
    ch = 2000

    bd = jnp.kron(jnp.eye(PK, dtype=jnp.float32),
                  jnp.full((de, de), 1.0 / de, jnp.float32))
    wblk = jnp.kron(jnp.eye(PK, dtype=jnp.float32),
                    jnp.eye(de, dtype=jnp.float32) + w_ee)
    bblk = jnp.tile(b_e, (PK,))
    efp = edge_features.reshape(e // PK, de * PK)

    ps, pt = _tc_project(node_features, w_es, w_et)
    gp = _sc_gather(ps, pt, src, tgt, epw, ch)
    nep = _tc_edge_ln(efp, gp, wblk, bd, bblk, g_e, bt_e, bp=2000)
    sum_p, cnt_p = _sc_scatter(nep, src, tgt, n, de, epw, ch)
    new_nodes = _tc_node_update(node_features, sum_p, cnt_p, w_n1, w_n2,
                                b_n, g_n, bt_n)
    return (new_nodes, nep.reshape(e, de))


# speedup vs baseline: 11.7305x; 1.0330x over previous
import functools

import jax
import jax.numpy as jnp
from jax import lax
from jax.experimental import pallas as pl
from jax.experimental.pallas import tpu as pltpu
from jax.experimental.pallas import tpu_sc as plsc

NC = 2
NS = 16
NW = NC * NS
PK = 8


def _proj_body(nf_ref, ws_ref, wt_ref, ps_ref, pt_ref):
    x = nf_ref[...]
    ps_ref[...] = jnp.dot(x, ws_ref[...], preferred_element_type=jnp.float32)
    pt_ref[...] = jnp.dot(x, wt_ref[...], preferred_element_type=jnp.float32)


def _tc_project(nf, w_s, w_t):
    n, _ = nf.shape
    de = w_s.shape[1]
    out = jax.ShapeDtypeStruct((n, de), jnp.float32)
    return pl.pallas_call(_proj_body, out_shape=(out, out))(nf, w_s, w_t)


def _sc_gather(ps, pt, ei, epw, ch):
    n, de = ps.shape
    e = ei.shape[1]
    ep, dep = e // PK, de * PK
    nchunk = epw // ch
    chp = ch // PK
    mesh = plsc.VectorSubcoreMesh(core_axis_name="c", subcore_axis_name="s")

    @functools.partial(
        pl.kernel,
        out_type=jax.ShapeDtypeStruct((ep, dep), jnp.float32),
        mesh=mesh,
        compiler_params=pltpu.CompilerParams(use_tc_tiling_on_sc=False),
        scratch_types=[
            pltpu.VMEM_SHARED((n, de), jnp.float32),
            pltpu.VMEM_SHARED((n, de), jnp.float32),
            pltpu.VMEM((ch,), jnp.int32),
            pltpu.VMEM((ch,), jnp.int32),
            pltpu.VMEM((ch, de), jnp.float32),
            pltpu.VMEM((ch, de), jnp.float32),
            pltpu.VMEM((chp, dep), jnp.float32),
        ],
    )
    def k(ps_hbm, pt_hbm, ei_hbm, out_hbm,
          ps_sh, pt_sh, idx_s, idx_t, gs, gt, xp):
        cid = lax.axis_index("c")
        sid = lax.axis_index("s")
        wid = cid * NS + sid

        @pl.when(sid == 0)
        def _():
            pltpu.sync_copy(ps_hbm, ps_sh)
            pltpu.sync_copy(pt_hbm, pt_sh)

        plsc.subcore_barrier()

        def chunk(ci, carry):
            base = wid * epw + ci * ch
            pbase = base // PK
            pltpu.sync_copy(ei_hbm.at[0, pl.ds(base, ch)], idx_s)
            pltpu.sync_copy(ei_hbm.at[1, pl.ds(base, ch)], idx_t)
            pltpu.sync_copy(ps_sh.at[idx_s], gs)
            pltpu.sync_copy(pt_sh.at[idx_t], gt)

            def row(i, c2):
                xp[i // PK, pl.ds((i % PK) * de, de)] = gs[i, :] + gt[i, :]
                return c2

            lax.fori_loop(0, ch, row, 0, unroll=8)
            pltpu.sync_copy(xp, out_hbm.at[pl.ds(pbase, chp)])
            return carry

        lax.fori_loop(0, nchunk, chunk, 0)

    return k(ps, pt, ei)


def _ln_body(efp_ref, gp_ref, wblk_ref, bd_ref, bblk_ref, ge_ref, bte_ref, out_ref):
    x = jnp.dot(efp_ref[...], wblk_ref[...], preferred_element_type=jnp.float32)
    x = x + gp_ref[...] + bblk_ref[...]
    mu = jnp.dot(x, bd_ref[...], preferred_element_type=jnp.float32)
    xc = x - mu
    var = jnp.dot(xc * xc, bd_ref[...], preferred_element_type=jnp.float32)
    out_ref[...] = xc * lax.rsqrt(var + 1e-5) * ge_ref[...] + bte_ref[...]


def _tc_edge_ln(efp, gp, wblk, bd, bblk, g_e, bt_e, bp):
    ep, dep = efp.shape
    grid = ep // bp
    row_spec = pl.BlockSpec((bp, dep), lambda i: (i, 0))
    par_spec = pl.BlockSpec((1, dep), lambda i: (0, 0))
    mat_spec = pl.BlockSpec((dep, dep), lambda i: (0, 0))
    return pl.pallas_call(
        _ln_body,
        grid=(grid,),
        in_specs=[row_spec, row_spec, mat_spec, mat_spec,
                  par_spec, par_spec, par_spec],
        out_specs=row_spec,
        out_shape=jax.ShapeDtypeStruct((ep, dep), jnp.float32),
    )(efp, gp, wblk, bd, bblk.reshape(1, dep),
      jnp.tile(g_e, (PK,)).reshape(1, dep),
      jnp.tile(bt_e, (PK,)).reshape(1, dep))


def _sc_scatter(ne2, ei, n, de, epw, ch):
    e = ne2.shape[0]
    nchunk = epw // ch
    mesh = plsc.VectorSubcoreMesh(core_axis_name="c", subcore_axis_name="s")
    zr = n // NS
    part = jax.ShapeDtypeStruct((NC, n, de), jnp.float32)

    @functools.partial(
        pl.kernel,
        out_type=(part, part),
        mesh=mesh,
        compiler_params=pltpu.CompilerParams(use_tc_tiling_on_sc=False),
        scratch_types=[
            pltpu.VMEM_SHARED((n, de), jnp.float32),
            pltpu.VMEM_SHARED((n, de), jnp.float32),
            pltpu.VMEM((ch,), jnp.int32),
            pltpu.VMEM((ch,), jnp.int32),
            pltpu.VMEM((ch, de), jnp.float32),
            pltpu.VMEM((ch, de), jnp.float32),
        ],
    )
    def k(ne_hbm, ei_hbm, zeros_hbm, ones_hbm,
          sum_hbm, cnt_hbm, acc_s, acc_c, idx_s, idx_t, val, ones_v):
        cid = lax.axis_index("c")
        sid = lax.axis_index("s")
        wid = cid * NS + sid

        pltpu.sync_copy(ones_hbm, ones_v)
        pltpu.sync_copy(zeros_hbm.at[pl.ds(sid * zr, zr)], acc_s.at[pl.ds(sid * zr, zr)])
        pltpu.sync_copy(zeros_hbm.at[pl.ds(sid * zr, zr)], acc_c.at[pl.ds(sid * zr, zr)])
        plsc.subcore_barrier()

        def chunk(ci, carry):
            base = wid * epw + ci * ch
            pltpu.sync_copy(ei_hbm.at[0, pl.ds(base, ch)], idx_s)
            pltpu.sync_copy(ei_hbm.at[1, pl.ds(base, ch)], idx_t)
            pltpu.sync_copy(ne_hbm.at[pl.ds(base, ch)], val)
            pltpu.sync_copy(val, acc_s.at[idx_s], add=True)
            pltpu.sync_copy(val, acc_s.at[idx_t], add=True)
            pltpu.sync_copy(ones_v, acc_c.at[idx_s], add=True)
            pltpu.sync_copy(ones_v, acc_c.at[idx_t], add=True)
            return carry

        lax.fori_loop(0, nchunk, chunk, 0)
        plsc.subcore_barrier()
        pltpu.sync_copy(acc_s.at[pl.ds(sid * zr, zr)], sum_hbm.at[cid, pl.ds(sid * zr, zr)])
        pltpu.sync_copy(acc_c.at[pl.ds(sid * zr, zr)], cnt_hbm.at[cid, pl.ds(sid * zr, zr)])

    zeros = jnp.zeros((n, de), jnp.float32)
    ones = jnp.ones((ch, de), jnp.float32)
    return k(ne2, ei, zeros, ones)


def _node_body(nf_ref, sum_ref, cnt_ref, wn1_ref, wn2_ref, bn_ref, gn_ref,
               btn_ref, out_ref):
    nf = nf_ref[...]
    s = sum_ref[0] + sum_ref[1]
    c = cnt_ref[0, :, 0:1] + cnt_ref[1, :, 0:1]
    m = s / (c + 1e-10)
    x = nf + jnp.dot(nf, wn1_ref[...], preferred_element_type=jnp.float32)
    x = x + jnp.dot(m, wn2_ref[...], preferred_element_type=jnp.float32)
    x = x + bn_ref[...]
    mu = jnp.mean(x, axis=-1, keepdims=True)
    xc = x - mu
    var = jnp.mean(xc * xc, axis=-1, keepdims=True)
    out_ref[...] = xc * lax.rsqrt(var + 1e-5) * gn_ref[...] + btn_ref[...]


def _tc_node_update(nf, sum_p, cnt_p, w_n1, w_n2, b_n, g_n, bt_n):
    n, dn = nf.shape
    return pl.pallas_call(
        _node_body,
        out_shape=jax.ShapeDtypeStruct((n, dn), jnp.float32),
    )(nf, sum_p, cnt_p, w_n1, w_n2,
      b_n.reshape(1, dn), g_n.reshape(1, dn), bt_n.reshape(1, dn))


def kernel(node_features, edge_features, edge_index, W_e, b_e, g_e, bt_e,
           W_n, b_n, g_n, bt_n):
    n, dn = node_features.shape
    e, de = edge_features.shape
    w_ee = W_e[:de]
    w_es = W_e[de:de + dn]
    w_et = W_e[de + dn:]
    w_n1 = W_n[:dn]
    w_n2 = W_n[dn:]

    epw = e // NW
    ch = 2000

    bd = jnp.kron(jnp.eye(PK, dtype=jnp.float32),
                  jnp.full((de, de), 1.0 / de, jnp.float32))
    wblk = jnp.kron(jnp.eye(PK, dtype=jnp.float32),
                    jnp.eye(de, dtype=jnp.float32) + w_ee)
    bblk = jnp.tile(b_e, (PK,))
    efp = edge_features.reshape(e // PK, de * PK)

    ps, pt = _tc_project(node_features, w_es, w_et)
    gp = _sc_gather(ps, pt, edge_index, epw, ch)
    nep = _tc_edge_ln(efp, gp, wblk, bd, bblk, g_e, bt_e, bp=2000)
    new_edges = nep.reshape(e, de)
    sum_p, cnt_p = _sc_scatter(new_edges, edge_index, n, de, epw, ch)
    new_nodes = _tc_node_update(node_features, sum_p, cnt_p, w_n1, w_n2,
                                b_n, g_n, bt_n)
    return (new_nodes, new_edges)
